# baseline (speedup 1.0000x reference)
import jax
import jax.numpy as jnp
from jax.experimental import pallas as pl
from jax.experimental.pallas import tpu as pltpu

_GPB = 2
_NSLICE = 2


def _gcn2_body(*refs):
    x_ref, w1_ref, b1_ref, w2_ref, b2_ref = refs[:5]
    a_refs = refs[5:5 + _NSLICE]
    o_ref = refs[5 + _NSLICE]

    n = x_ref.shape[1]
    w = n // _NSLICE
    eye = (jax.lax.broadcasted_iota(jnp.int32, (w, w), 0)
           == jax.lax.broadcasted_iota(jnp.int32, (w, w), 1)
           ).astype(jnp.float32)

    ones_row = jnp.ones((1, w), jnp.bfloat16)
    for g in range(_GPB):
        asb = [a_refs[j][g].astype(jnp.bfloat16) for j in range(_NSLICE)]
        cs = sum(
            jax.lax.dot_general(ones_row, asb[j], (((1,), (0,)), ((), ())),
                                preferred_element_type=jnp.float32)
            for j in range(_NSLICE)
        )[0]
        diag = jnp.concatenate([
            jnp.sum(asb[j][:, j * w:(j + 1) * w].astype(jnp.float32) * eye,
                    axis=0)
            for j in range(_NSLICE)
        ])
        mask = (diag == 0.0).astype(jnp.float32)
        deg = cs + mask
        dinv = jnp.where(deg > 0.0, jax.lax.rsqrt(deg), 0.0)
        dcol = dinv[:, None]
        md = mask[:, None] * dcol

        def ahT_dot(yb):
            return sum(
                jax.lax.dot_general(asb[j], yb[j * w:(j + 1) * w],
                                    (((0,), (0,)), ((), ())),
                                    preferred_element_type=jnp.float32)
                for j in range(_NSLICE)
            )

        xw = jnp.dot(x_ref[g], w1_ref[...], preferred_element_type=jnp.float32)
        y = dcol * xw
        t = ahT_dot(y.astype(jnp.bfloat16)) + mask[:, None] * y
        h = jnp.maximum(dcol * t + b1_ref[0], 0.0)

        hw = jnp.dot(h, w2_ref[...], preferred_element_type=jnp.float32)
        y2 = dcol * hw
        t2 = ahT_dot(y2.astype(jnp.bfloat16)) + md * hw
        o_ref[g] = dcol * t2 + b2_ref[0]


def kernel(x, A, W1, b1, W2, b2):
    Bb, n, in_c = x.shape
    hid = W1.shape[1]
    out_c = W2.shape[1]
    w = n // _NSLICE

    a_specs = [
        pl.BlockSpec((_GPB, w, n), lambda i, j=j: (i, j, 0))
        for j in range(_NSLICE)
    ]
    return pl.pallas_call(
        _gcn2_body,
        grid=(Bb // _GPB,),
        in_specs=[
            pl.BlockSpec((_GPB, n, in_c), lambda i: (i, 0, 0)),
            pl.BlockSpec((in_c, hid), lambda i: (0, 0)),
            pl.BlockSpec((1, hid), lambda i: (0, 0)),
            pl.BlockSpec((hid, out_c), lambda i: (0, 0)),
            pl.BlockSpec((1, out_c), lambda i: (0, 0)),
        ] + a_specs,
        out_specs=pl.BlockSpec((_GPB, n, out_c), lambda i: (i, 0, 0)),
        out_shape=jax.ShapeDtypeStruct((Bb, n, out_c), jnp.float32),
        compiler_params=pltpu.CompilerParams(
            dimension_semantics=("parallel",)),
    )(x, W1, b1.reshape(1, hid), W2, b2.reshape(1, out_c), *([A] * _NSLICE))


# speedup vs baseline: 1.0612x; 1.0612x over previous
import jax
import jax.numpy as jnp
from jax.experimental import pallas as pl
from jax.experimental.pallas import tpu as pltpu

_GPB = 2
_NSLICE = 4


def _gcn2_body(*refs):
    x_ref, w1_ref, b1_ref, w2_ref, b2_ref = refs[:5]
    a_refs = refs[5:5 + _NSLICE]
    o_ref = refs[5 + _NSLICE]

    n = x_ref.shape[1]
    w = n // _NSLICE
    eye = (jax.lax.broadcasted_iota(jnp.int32, (w, w), 0)
           == jax.lax.broadcasted_iota(jnp.int32, (w, w), 1)
           ).astype(jnp.float32)

    ones_row = jnp.ones((1, w), jnp.bfloat16)
    for g in range(_GPB):
        asb = [a_refs[j][g].astype(jnp.bfloat16) for j in range(_NSLICE)]
        cs = sum(
            jax.lax.dot_general(ones_row, asb[j], (((1,), (0,)), ((), ())),
                                preferred_element_type=jnp.float32)
            for j in range(_NSLICE)
        )[0]
        diag = jnp.concatenate([
            jnp.sum(asb[j][:, j * w:(j + 1) * w].astype(jnp.float32) * eye,
                    axis=0)
            for j in range(_NSLICE)
        ])
        mask = (diag == 0.0).astype(jnp.float32)
        deg = cs + mask
        dinv = jnp.where(deg > 0.0, jax.lax.rsqrt(deg), 0.0)
        dcol = dinv[:, None]
        md = mask[:, None] * dcol

        def ahT_dot(yb):
            return sum(
                jax.lax.dot_general(asb[j], yb[j * w:(j + 1) * w],
                                    (((0,), (0,)), ((), ())),
                                    preferred_element_type=jnp.float32)
                for j in range(_NSLICE)
            )

        xw = jnp.dot(x_ref[g], w1_ref[...], preferred_element_type=jnp.float32)
        y = dcol * xw
        t = ahT_dot(y.astype(jnp.bfloat16)) + mask[:, None] * y
        h = jnp.maximum(dcol * t + b1_ref[0], 0.0)

        hw = jnp.dot(h, w2_ref[...], preferred_element_type=jnp.float32)
        y2 = dcol * hw
        t2 = ahT_dot(y2.astype(jnp.bfloat16)) + md * hw
        o_ref[g] = dcol * t2 + b2_ref[0]


def kernel(x, A, W1, b1, W2, b2):
    Bb, n, in_c = x.shape
    hid = W1.shape[1]
    out_c = W2.shape[1]
    w = n // _NSLICE

    a_specs = [
        pl.BlockSpec((_GPB, w, n), lambda i, j=j: (i, j, 0))
        for j in range(_NSLICE)
    ]
    return pl.pallas_call(
        _gcn2_body,
        grid=(Bb // _GPB,),
        in_specs=[
            pl.BlockSpec((_GPB, n, in_c), lambda i: (i, 0, 0)),
            pl.BlockSpec((in_c, hid), lambda i: (0, 0)),
            pl.BlockSpec((1, hid), lambda i: (0, 0)),
            pl.BlockSpec((hid, out_c), lambda i: (0, 0)),
            pl.BlockSpec((1, out_c), lambda i: (0, 0)),
        ] + a_specs,
        out_specs=pl.BlockSpec((_GPB, n, out_c), lambda i: (i, 0, 0)),
        out_shape=jax.ShapeDtypeStruct((Bb, n, out_c), jnp.float32),
        compiler_params=pltpu.CompilerParams(
            dimension_semantics=("parallel",)),
    )(x, W1, b1.reshape(1, hid), W2, b2.reshape(1, out_c), *([A] * _NSLICE))
